# baseline (speedup 1.0000x reference)
import jax
import jax.numpy as jnp
import numpy as np
from jax import lax
from jax.experimental import pallas as pl
from jax.experimental.pallas import tpu as pltpu
from jax.experimental.pallas import tpu_sc as plsc

B, T, D, A = 4, 8192, 768, 256
RINGS = 1
TB = 1024
NT = T // TB

NC, NS, L = 2, 16, 16
NW = NC * NS
TCUT = 6144
STRIDE = (T - TCUT) // (NW // B)
CH = 16
NCHUNK = STRIDE // CH
NSLOT = 4


def _neighbor_w():
    w = np.zeros((A, A), dtype=np.float32)
    for a in range(A):
        lo, hi = max(0, a - RINGS), min(A, a + RINGS + 1)
        w[a, lo:hi] = 1.0 / (hi - lo)
    return jnp.asarray(w)


def _pool_body(assign_ref, x_ref, sums_ref, counts_ref):
    b = pl.program_id(0)
    tb = pl.program_id(1)
    a_ids = assign_ref[0, 0, :]
    rows = lax.broadcasted_iota(jnp.int32, (A, TB), 0)
    onehot_t = (rows == a_ids[None, :]).astype(jnp.bfloat16)
    x_blk = x_ref[0].astype(jnp.bfloat16)
    partial = lax.dot_general(
        onehot_t, x_blk, (((1,), (0,)), ((), ())),
        preferred_element_type=jnp.float32)

    @pl.when(tb == 0)
    def _():
        sums_ref[0] = partial

    @pl.when(tb != 0)
    def _():
        sums_ref[0] += partial

    cpart = jnp.sum((rows == a_ids[None, :]).astype(jnp.float32), axis=1)

    @pl.when((b == 0) & (tb == 0))
    def _():
        counts_ref[0, :] = cpart

    @pl.when((b == 0) & (tb != 0))
    def _():
        counts_ref[0, :] += cpart


def _mid_body(gate_ref, hops_ref, sums_ref, counts_ref, wn_ref, wp_ref,
              scaled_ref):
    inv = 1.0 / jnp.maximum(counts_ref[0, :], 1.0)
    g = jnp.tanh(gate_ref[0])
    wn = wn_ref[...]
    wp = wp_ref[...]
    nhops = jnp.maximum(1, hops_ref[0])
    for b in range(B):
        anchor = sums_ref[b] * inv[:, None]
        agg = lax.fori_loop(
            0, nhops,
            lambda _, a: jnp.dot(wn, a, preferred_element_type=jnp.float32),
            anchor)
        proj = lax.dot_general(
            agg, wp, (((1,), (1,)), ((), ())),
            preferred_element_type=jnp.float32)
        scaled_ref[b] = proj * g


def _bcast_tc_body(assign_ref, x_ref, scaled_ref, out_ref):
    a_ids = assign_ref[0, 0, :]
    cols = lax.broadcasted_iota(jnp.int32, (TB, A), 1)
    onehot = (cols == a_ids[:, None]).astype(jnp.bfloat16)
    fb = lax.dot_general(
        onehot, scaled_ref[0].astype(jnp.bfloat16), (((1,), (0,)), ((), ())),
        preferred_element_type=jnp.float32)
    out_ref[0] = x_ref[0] + fb


def _bcast_sc_body(x_ref, assign_ref, scaled_ref, out_ref, idx_all, *scratch):
    xbs = scratch[0:NSLOT]
    gbs = scratch[NSLOT:2 * NSLOT]
    sxs = scratch[2 * NSLOT:3 * NSLOT]
    sgs = scratch[3 * NSLOT:4 * NSLOT]
    sts = scratch[4 * NSLOT:5 * NSLOT]
    c = lax.axis_index("c")
    s = lax.axis_index("s")
    wid = s * NC + c
    b = wid // (NW // B)
    lane = wid % (NW // B)
    tok0 = TCUT + lane * STRIDE
    row0 = b * T + tok0
    boff = b * A

    pltpu.sync_copy(assign_ref.at[pl.ds(tok0, STRIDE)], idx_all)

    def _off(k, _):
        sl = pl.ds(k * L, L)
        idx_all[sl] = idx_all[sl] + boff
        return 0

    lax.fori_loop(0, STRIDE // L, _off, 0)

    def issue_loads(j, q):
        pltpu.async_copy(x_ref.at[pl.ds(row0 + j * CH, CH)], xbs[q], sxs[q])
        pltpu.async_copy(
            scaled_ref.at[plsc.Indices(idx_all.at[pl.ds(j * CH, CH)])],
            gbs[q], sgs[q])

    issue_loads(0, 0)
    issue_loads(1, 1)

    def chunk_quad(j4, _):
        j0 = NSLOT * j4
        for u in range(NSLOT):
            j = j0 + u
            xb, gb, sx, sg, st = xbs[u], gbs[u], sxs[u], sgs[u], sts[u]
            pltpu.make_async_copy(
                x_ref.at[pl.ds(row0 + j * CH, CH)], xb, sx).wait()
            pltpu.make_async_copy(
                scaled_ref.at[plsc.Indices(idx_all.at[pl.ds(j * CH, CH)])],
                gb, sg).wait()

            def _add(r, _):
                for k in range(D // L):
                    sl = pl.ds(k * L, L)
                    xb[r, sl] = xb[r, sl] + gb[r, sl]
                return 0

            lax.fori_loop(0, CH, _add, 0)
            pltpu.async_copy(xb, out_ref.at[pl.ds(row0 + j * CH, CH)], st)

            qn = (u + 2) % NSLOT

            @pl.when(j >= 2)
            def _():
                pltpu.make_async_copy(
                    xbs[qn],
                    out_ref.at[pl.ds(row0 + (j - 2) * CH, CH)],
                    sts[qn]).wait()

            @pl.when(j + 2 < NCHUNK)
            def _():
                jn = j + 2
                pltpu.async_copy(
                    x_ref.at[pl.ds(row0 + jn * CH, CH)], xbs[qn], sxs[qn])
                pltpu.async_copy(
                    scaled_ref.at[
                        plsc.Indices(idx_all.at[pl.ds(jn * CH, CH)])],
                    gbs[qn], sgs[qn])

        return 0

    lax.fori_loop(0, NCHUNK // NSLOT, chunk_quad, 0)

    for j in (NCHUNK - 2, NCHUNK - 1):
        q = j % NSLOT
        pltpu.make_async_copy(
            xbs[q], out_ref.at[pl.ds(row0 + j * CH, CH)], sts[q]).wait()


def _bcast_sc(x_flat, assign_flat, scaled_flat):
    mesh = plsc.VectorSubcoreMesh(core_axis_name="c", subcore_axis_name="s")
    kfun = pl.kernel(
        _bcast_sc_body,
        out_type=jax.ShapeDtypeStruct((B * T, D), jnp.float32),
        mesh=mesh,
        scratch_types=(
            [pltpu.VMEM((STRIDE,), jnp.int32)]
            + [pltpu.VMEM((CH, D), jnp.float32) for _ in range(2 * NSLOT)]
            + [pltpu.SemaphoreType.DMA for _ in range(3 * NSLOT)]
        ),
    )
    return kfun(x_flat, assign_flat, scaled_flat)


def kernel(x, assign, W_proj, gate, hops):
    assign_i = assign.astype(jnp.int32)
    assign3 = assign_i.reshape(NT, 1, TB)

    sums, counts = pl.pallas_call(
        _pool_body,
        grid=(B, NT),
        in_specs=[
            pl.BlockSpec((1, 1, TB), lambda b, t: (t, 0, 0)),
            pl.BlockSpec((1, TB, D), lambda b, t: (b, t, 0)),
        ],
        out_specs=[
            pl.BlockSpec((1, A, D), lambda b, t: (b, 0, 0)),
            pl.BlockSpec((1, A), lambda b, t: (0, 0)),
        ],
        out_shape=[
            jax.ShapeDtypeStruct((B, A, D), jnp.float32),
            jax.ShapeDtypeStruct((1, A), jnp.float32),
        ],
    )(assign3, x)

    wn = _neighbor_w()
    gate_s = jnp.reshape(jnp.asarray(gate, jnp.float32), (1,))
    hops_s = jnp.reshape(jnp.asarray(hops, jnp.int32), (1,))
    scaled = pl.pallas_call(
        _mid_body,
        in_specs=[
            pl.BlockSpec(memory_space=pltpu.SMEM),
            pl.BlockSpec(memory_space=pltpu.SMEM),
            pl.BlockSpec((B, A, D), lambda: (0, 0, 0)),
            pl.BlockSpec((1, A), lambda: (0, 0)),
            pl.BlockSpec((A, A), lambda: (0, 0)),
            pl.BlockSpec((D, D), lambda: (0, 0)),
        ],
        out_specs=pl.BlockSpec((B, A, D), lambda: (0, 0, 0)),
        out_shape=jax.ShapeDtypeStruct((B, A, D), jnp.float32),
    )(gate_s, hops_s, sums, counts, wn, W_proj)

    out_sc = _bcast_sc(x.reshape(B * T, D), assign_i,
                       scaled.reshape(B * A, D)).reshape(B, T, D)

    if TCUT > 0:
        ntc = TCUT // TB
        out_tc = pl.pallas_call(
            _bcast_tc_body,
            grid=(B, ntc),
            in_specs=[
                pl.BlockSpec((1, 1, TB), lambda b, t: (t, 0, 0)),
                pl.BlockSpec((1, TB, D), lambda b, t: (b, t, 0)),
                pl.BlockSpec((1, A, D), lambda b, t: (b, 0, 0)),
            ],
            out_specs=pl.BlockSpec((1, TB, D), lambda b, t: (b, t, 0)),
            out_shape=jax.ShapeDtypeStruct((B, TCUT, D), jnp.float32),
        )(assign3[:ntc], x[:, :TCUT], scaled)
        out = jnp.concatenate([out_tc, out_sc[:, TCUT:]], axis=1)
    else:
        out = out_sc

    return out


# speedup vs baseline: 1.7654x; 1.7654x over previous
import jax
import jax.numpy as jnp
import numpy as np
from jax import lax
from jax.experimental import pallas as pl
from jax.experimental.pallas import tpu as pltpu
from jax.experimental.pallas import tpu_sc as plsc

B, T, D, A = 4, 8192, 768, 256
RINGS = 1
TB = 1024
NT = T // TB

NC, NS, L = 2, 16, 16
NW = NC * NS
TCUT = 7168
OUT_T = T - TCUT
STRIDE = (T - TCUT) // (NW // B)
CH = 16
NCHUNK = STRIDE // CH
NSLOT = 4


def _neighbor_w():
    w = np.zeros((A, A), dtype=np.float32)
    for a in range(A):
        lo, hi = max(0, a - RINGS), min(A, a + RINGS + 1)
        w[a, lo:hi] = 1.0 / (hi - lo)
    return jnp.asarray(w)


def _pool_body(assign_ref, x_ref, sums_ref, counts_ref):
    b = pl.program_id(0)
    tb = pl.program_id(1)
    a_ids = assign_ref[0, 0, :]
    rows = lax.broadcasted_iota(jnp.int32, (A, TB), 0)
    onehot_t = (rows == a_ids[None, :]).astype(jnp.bfloat16)
    x_blk = x_ref[0].astype(jnp.bfloat16)
    partial = lax.dot_general(
        onehot_t, x_blk, (((1,), (0,)), ((), ())),
        preferred_element_type=jnp.float32)

    @pl.when(tb == 0)
    def _():
        sums_ref[0] = partial

    @pl.when(tb != 0)
    def _():
        sums_ref[0] += partial

    cpart = jnp.sum((rows == a_ids[None, :]).astype(jnp.float32), axis=1)

    @pl.when((b == 0) & (tb == 0))
    def _():
        counts_ref[0, :] = cpart

    @pl.when((b == 0) & (tb != 0))
    def _():
        counts_ref[0, :] += cpart


def _mid_body(gate_ref, hops_ref, sums_ref, counts_ref, wn_ref, wp_ref,
              scaled_ref):
    inv = 1.0 / jnp.maximum(counts_ref[0, :], 1.0)
    g = jnp.tanh(gate_ref[0])
    wn = wn_ref[...]
    wp = wp_ref[...]
    nhops = jnp.maximum(1, hops_ref[0])
    for b in range(B):
        anchor = sums_ref[b] * inv[:, None]
        agg = lax.fori_loop(
            0, nhops,
            lambda _, a: jnp.dot(wn, a, preferred_element_type=jnp.float32),
            anchor)
        proj = lax.dot_general(
            agg, wp, (((1,), (1,)), ((), ())),
            preferred_element_type=jnp.float32)
        scaled_ref[b] = proj * g


def _bcast_tc_body(assign_ref, x_ref, scaled_ref, out_ref):
    a_ids = assign_ref[0, 0, :]
    cols = lax.broadcasted_iota(jnp.int32, (TB, A), 1)
    onehot = (cols == a_ids[:, None]).astype(jnp.bfloat16)
    fb = lax.dot_general(
        onehot, scaled_ref[0].astype(jnp.bfloat16), (((1,), (0,)), ((), ())),
        preferred_element_type=jnp.float32)
    out_ref[0] = x_ref[0] + fb


def _bcast_sc_body(x_ref, assign_ref, scaled_ref, out_ref, idx_all, *scratch):
    xbs = scratch[0:NSLOT]
    gbs = scratch[NSLOT:2 * NSLOT]
    sxs = scratch[2 * NSLOT:3 * NSLOT]
    sgs = scratch[3 * NSLOT:4 * NSLOT]
    sts = scratch[4 * NSLOT:5 * NSLOT]
    c = lax.axis_index("c")
    s = lax.axis_index("s")
    wid = s * NC + c
    b = wid // (NW // B)
    lane = wid % (NW // B)
    tok0 = TCUT + lane * STRIDE
    row0 = b * T + tok0
    orow0 = b * OUT_T + lane * STRIDE
    boff = b * A

    pltpu.sync_copy(assign_ref.at[pl.ds(tok0, STRIDE)], idx_all)

    def _off(k, _):
        sl = pl.ds(k * L, L)
        idx_all[sl] = idx_all[sl] + boff
        return 0

    lax.fori_loop(0, STRIDE // L, _off, 0)

    def issue_loads(j, q):
        pltpu.async_copy(x_ref.at[pl.ds(row0 + j * CH, CH)], xbs[q], sxs[q])
        pltpu.async_copy(
            scaled_ref.at[plsc.Indices(idx_all.at[pl.ds(j * CH, CH)])],
            gbs[q], sgs[q])

    issue_loads(0, 0)
    issue_loads(1, 1)

    def chunk_quad(j4, _):
        j0 = NSLOT * j4
        for u in range(NSLOT):
            j = j0 + u
            xb, gb, sx, sg, st = xbs[u], gbs[u], sxs[u], sgs[u], sts[u]
            pltpu.make_async_copy(
                x_ref.at[pl.ds(row0 + j * CH, CH)], xb, sx).wait()
            pltpu.make_async_copy(
                scaled_ref.at[plsc.Indices(idx_all.at[pl.ds(j * CH, CH)])],
                gb, sg).wait()

            def _add(r, _):
                for k in range(D // L):
                    sl = pl.ds(k * L, L)
                    xb[r, sl] = xb[r, sl] + gb[r, sl]
                return 0

            lax.fori_loop(0, CH, _add, 0)
            pltpu.async_copy(xb, out_ref.at[pl.ds(orow0 + j * CH, CH)], st)

            qn = (u + 2) % NSLOT

            @pl.when(j >= 2)
            def _():
                pltpu.make_async_copy(
                    xbs[qn],
                    out_ref.at[pl.ds(orow0 + (j - 2) * CH, CH)],
                    sts[qn]).wait()

            @pl.when(j + 2 < NCHUNK)
            def _():
                jn = j + 2
                pltpu.async_copy(
                    x_ref.at[pl.ds(row0 + jn * CH, CH)], xbs[qn], sxs[qn])
                pltpu.async_copy(
                    scaled_ref.at[
                        plsc.Indices(idx_all.at[pl.ds(jn * CH, CH)])],
                    gbs[qn], sgs[qn])

        return 0

    lax.fori_loop(0, NCHUNK // NSLOT, chunk_quad, 0)

    for j in (NCHUNK - 2, NCHUNK - 1):
        q = j % NSLOT
        pltpu.make_async_copy(
            xbs[q], out_ref.at[pl.ds(orow0 + j * CH, CH)], sts[q]).wait()


def _bcast_sc(x_flat, assign_flat, scaled_flat):
    mesh = plsc.VectorSubcoreMesh(core_axis_name="c", subcore_axis_name="s")
    kfun = pl.kernel(
        _bcast_sc_body,
        out_type=jax.ShapeDtypeStruct((B * OUT_T, D), jnp.float32),
        mesh=mesh,
        scratch_types=(
            [pltpu.VMEM((STRIDE,), jnp.int32)]
            + [pltpu.VMEM((CH, D), jnp.float32) for _ in range(2 * NSLOT)]
            + [pltpu.SemaphoreType.DMA for _ in range(3 * NSLOT)]
        ),
    )
    return kfun(x_flat, assign_flat, scaled_flat)


def kernel(x, assign, W_proj, gate, hops):
    assign_i = assign.astype(jnp.int32)
    assign3 = assign_i.reshape(NT, 1, TB)

    sums, counts = pl.pallas_call(
        _pool_body,
        grid=(B, NT),
        in_specs=[
            pl.BlockSpec((1, 1, TB), lambda b, t: (t, 0, 0)),
            pl.BlockSpec((1, TB, D), lambda b, t: (b, t, 0)),
        ],
        out_specs=[
            pl.BlockSpec((1, A, D), lambda b, t: (b, 0, 0)),
            pl.BlockSpec((1, A), lambda b, t: (0, 0)),
        ],
        out_shape=[
            jax.ShapeDtypeStruct((B, A, D), jnp.float32),
            jax.ShapeDtypeStruct((1, A), jnp.float32),
        ],
    )(assign3, x)

    wn = _neighbor_w()
    gate_s = jnp.reshape(jnp.asarray(gate, jnp.float32), (1,))
    hops_s = jnp.reshape(jnp.asarray(hops, jnp.int32), (1,))
    scaled = pl.pallas_call(
        _mid_body,
        in_specs=[
            pl.BlockSpec(memory_space=pltpu.SMEM),
            pl.BlockSpec(memory_space=pltpu.SMEM),
            pl.BlockSpec((B, A, D), lambda: (0, 0, 0)),
            pl.BlockSpec((1, A), lambda: (0, 0)),
            pl.BlockSpec((A, A), lambda: (0, 0)),
            pl.BlockSpec((D, D), lambda: (0, 0)),
        ],
        out_specs=pl.BlockSpec((B, A, D), lambda: (0, 0, 0)),
        out_shape=jax.ShapeDtypeStruct((B, A, D), jnp.float32),
    )(gate_s, hops_s, sums, counts, wn, W_proj)

    out_sc = _bcast_sc(x.reshape(B * T, D), assign_i,
                       scaled.reshape(B * A, D)).reshape(B, OUT_T, D)

    ntc = TCUT // TB
    out_tc = pl.pallas_call(
        _bcast_tc_body,
        grid=(B, ntc),
        in_specs=[
            pl.BlockSpec((1, 1, TB), lambda b, t: (t, 0, 0)),
            pl.BlockSpec((1, TB, D), lambda b, t: (b, t, 0)),
            pl.BlockSpec((1, A, D), lambda b, t: (b, 0, 0)),
        ],
        out_specs=pl.BlockSpec((1, TB, D), lambda b, t: (b, t, 0)),
        out_shape=jax.ShapeDtypeStruct((B, T, D), jnp.float32),
    )(assign3[:ntc], x, scaled)
    return lax.dynamic_update_slice(out_tc, out_sc, (0, TCUT, 0))


# speedup vs baseline: 1.8514x; 1.0487x over previous
import jax
import jax.numpy as jnp
import numpy as np
from jax import lax
from jax.experimental import pallas as pl
from jax.experimental.pallas import tpu as pltpu
from jax.experimental.pallas import tpu_sc as plsc

B, T, D, A = 4, 8192, 768, 256
RINGS = 1
TB = 1024
NT = T // TB

NC, NS, L = 2, 16, 16
NW = NC * NS
TCUT = 7168
OUT_T = T - TCUT
STRIDE = (T - TCUT) // (NW // B)
CH = 16
NCHUNK = STRIDE // CH
NSLOT = 4


def _neighbor_w():
    w = np.zeros((A, A), dtype=np.float32)
    for a in range(A):
        lo, hi = max(0, a - RINGS), min(A, a + RINGS + 1)
        w[a, lo:hi] = 1.0 / (hi - lo)
    return jnp.asarray(w)


def _poolmid_body(gate_ref, hops_ref, assign_ref, x_ref, wn_ref, wp_ref,
                  scaled_ref, sums_s, counts_s):
    b = pl.program_id(0)
    tb = pl.program_id(1)
    a_ids = assign_ref[0, 0, :]
    rows = lax.broadcasted_iota(jnp.int32, (A, TB), 0)
    onehot_t = (rows == a_ids[None, :]).astype(jnp.bfloat16)
    x_blk = x_ref[0].astype(jnp.bfloat16)
    partial = lax.dot_general(
        onehot_t, x_blk, (((1,), (0,)), ((), ())),
        preferred_element_type=jnp.float32)

    @pl.when(tb == 0)
    def _():
        sums_s[...] = partial

    @pl.when(tb != 0)
    def _():
        sums_s[...] += partial

    cpart = jnp.sum((rows == a_ids[None, :]).astype(jnp.float32), axis=1)

    @pl.when((b == 0) & (tb == 0))
    def _():
        counts_s[0, :] = cpart

    @pl.when((b == 0) & (tb != 0))
    def _():
        counts_s[0, :] += cpart

    @pl.when(tb == NT - 1)
    def _():
        inv = 1.0 / jnp.maximum(counts_s[0, :], 1.0)
        g = jnp.tanh(gate_ref[0])
        anchor = sums_s[...] * inv[:, None]
        agg = lax.fori_loop(
            0, jnp.maximum(1, hops_ref[0]),
            lambda _, a: jnp.dot(wn_ref[...], a,
                                 preferred_element_type=jnp.float32),
            anchor)
        proj = lax.dot_general(
            agg, wp_ref[...], (((1,), (1,)), ((), ())),
            preferred_element_type=jnp.float32)
        scaled_ref[0] = proj * g


def _bcast_sc_body(x_ref, assign_ref, scaled_ref, out_ref, idx_all, *scratch):
    xbs = scratch[0:NSLOT]
    gbs = scratch[NSLOT:2 * NSLOT]
    sxs = scratch[2 * NSLOT:3 * NSLOT]
    sgs = scratch[3 * NSLOT:4 * NSLOT]
    sts = scratch[4 * NSLOT:5 * NSLOT]
    c = lax.axis_index("c")
    s = lax.axis_index("s")
    wid = s * NC + c
    b = wid // (NW // B)
    lane = wid % (NW // B)
    tok0 = TCUT + lane * STRIDE
    row0 = b * T + tok0
    orow0 = row0
    boff = b * A

    pltpu.sync_copy(assign_ref.at[pl.ds(tok0, STRIDE)], idx_all)

    def _off(k, _):
        sl = pl.ds(k * L, L)
        idx_all[sl] = idx_all[sl] + boff
        return 0

    lax.fori_loop(0, STRIDE // L, _off, 0)

    def issue_loads(j, q):
        pltpu.async_copy(x_ref.at[pl.ds(row0 + j * CH, CH)], xbs[q], sxs[q])
        pltpu.async_copy(
            scaled_ref.at[plsc.Indices(idx_all.at[pl.ds(j * CH, CH)])],
            gbs[q], sgs[q])

    issue_loads(0, 0)
    issue_loads(1, 1)

    def chunk_quad(j4, _):
        j0 = NSLOT * j4
        for u in range(NSLOT):
            j = j0 + u
            xb, gb, sx, sg, st = xbs[u], gbs[u], sxs[u], sgs[u], sts[u]
            pltpu.make_async_copy(
                x_ref.at[pl.ds(row0 + j * CH, CH)], xb, sx).wait()
            pltpu.make_async_copy(
                scaled_ref.at[plsc.Indices(idx_all.at[pl.ds(j * CH, CH)])],
                gb, sg).wait()

            def _add(r, _):
                for k in range(D // L):
                    sl = pl.ds(k * L, L)
                    xb[r, sl] = xb[r, sl] + gb[r, sl]
                return 0

            lax.fori_loop(0, CH, _add, 0)
            pltpu.async_copy(xb, out_ref.at[pl.ds(orow0 + j * CH, CH)], st)

            qn = (u + 2) % NSLOT

            @pl.when(j >= 2)
            def _():
                pltpu.make_async_copy(
                    xbs[qn],
                    out_ref.at[pl.ds(orow0 + (j - 2) * CH, CH)],
                    sts[qn]).wait()

            @pl.when(j + 2 < NCHUNK)
            def _():
                jn = j + 2
                pltpu.async_copy(
                    x_ref.at[pl.ds(row0 + jn * CH, CH)], xbs[qn], sxs[qn])
                pltpu.async_copy(
                    scaled_ref.at[
                        plsc.Indices(idx_all.at[pl.ds(jn * CH, CH)])],
                    gbs[qn], sgs[qn])

        return 0

    lax.fori_loop(0, NCHUNK // NSLOT, chunk_quad, 0)

    for j in (NCHUNK - 2, NCHUNK - 1):
        q = j % NSLOT
        pltpu.make_async_copy(
            xbs[q], out_ref.at[pl.ds(orow0 + j * CH, CH)], sts[q]).wait()


def _bcast_sc(x_flat, assign_flat, scaled_flat):
    mesh = plsc.VectorSubcoreMesh(core_axis_name="c", subcore_axis_name="s")
    kfun = pl.kernel(
        _bcast_sc_body,
        out_type=jax.ShapeDtypeStruct((B * T, D), jnp.float32),
        mesh=mesh,
        scratch_types=(
            [pltpu.VMEM((STRIDE,), jnp.int32)]
            + [pltpu.VMEM((CH, D), jnp.float32) for _ in range(2 * NSLOT)]
            + [pltpu.SemaphoreType.DMA for _ in range(3 * NSLOT)]
        ),
    )
    return kfun(x_flat, assign_flat, scaled_flat)


def _bcast_tc_body(assign_ref, x_ref, scaled_ref, osc_ref, out_ref):
    del osc_ref
    a_ids = assign_ref[0, 0, :]
    cols = lax.broadcasted_iota(jnp.int32, (TB, A), 1)
    onehot = (cols == a_ids[:, None]).astype(jnp.bfloat16)
    fb = lax.dot_general(
        onehot, scaled_ref[0].astype(jnp.bfloat16), (((1,), (0,)), ((), ())),
        preferred_element_type=jnp.float32)
    out_ref[0] = x_ref[0] + fb


def kernel(x, assign, W_proj, gate, hops):
    assign_i = assign.astype(jnp.int32)
    assign3 = assign_i.reshape(NT, 1, TB)

    wn = _neighbor_w()
    gate_s = jnp.reshape(jnp.asarray(gate, jnp.float32), (1,))
    hops_s = jnp.reshape(jnp.asarray(hops, jnp.int32), (1,))
    scaled = pl.pallas_call(
        _poolmid_body,
        grid=(B, NT),
        in_specs=[
            pl.BlockSpec(memory_space=pltpu.SMEM),
            pl.BlockSpec(memory_space=pltpu.SMEM),
            pl.BlockSpec((1, 1, TB), lambda b, t: (t, 0, 0)),
            pl.BlockSpec((1, TB, D), lambda b, t: (b, t, 0)),
            pl.BlockSpec((A, A), lambda b, t: (0, 0)),
            pl.BlockSpec((D, D), lambda b, t: (0, 0)),
        ],
        out_specs=pl.BlockSpec((1, A, D), lambda b, t: (b, 0, 0)),
        out_shape=jax.ShapeDtypeStruct((B, A, D), jnp.float32),
        scratch_shapes=[
            pltpu.VMEM((A, D), jnp.float32),
            pltpu.VMEM((1, A), jnp.float32),
        ],
    )(gate_s, hops_s, assign3, x, wn, W_proj)

    out_sc = _bcast_sc(x.reshape(B * T, D), assign_i,
                       scaled.reshape(B * A, D)).reshape(B, T, D)

    ntc = TCUT // TB
    out = pl.pallas_call(
        _bcast_tc_body,
        grid=(B, ntc),
        in_specs=[
            pl.BlockSpec((1, 1, TB), lambda b, t: (t, 0, 0)),
            pl.BlockSpec((1, TB, D), lambda b, t: (b, t, 0)),
            pl.BlockSpec((1, A, D), lambda b, t: (b, 0, 0)),
            pl.BlockSpec(memory_space=pltpu.MemorySpace.HBM),
        ],
        out_specs=pl.BlockSpec((1, TB, D), lambda b, t: (b, t, 0)),
        out_shape=jax.ShapeDtypeStruct((B, T, D), jnp.float32),
        input_output_aliases={3: 0},
    )(assign3[:ntc], x, scaled, out_sc)
    return out
